# baseline (speedup 1.0000x reference)
import functools

import jax
import jax.numpy as jnp
from jax import lax
from jax.experimental import pallas as pl
from jax.experimental.pallas import tpu as pltpu
from jax.experimental.pallas import tpu_sc as plsc

_B = 4096
_D = 32
_R = 64
_NC = 2
_NS = 16
_NW = _NC * _NS
_BPW = _B // _NW

_BB = 512
_G = _B // _BB


def _sc_gather(users, pos_items, neg_items, heads, relations, pos_tails,
               neg_tails, user_embed, item_embed, kg_entity_embed,
               kg_relation_embed):
    mesh = plsc.VectorSubcoreMesh(core_axis_name="c", subcore_axis_name="s")
    out_t = [jax.ShapeDtypeStruct((_B, _D), jnp.float32)] * 9

    @functools.partial(
        pl.kernel,
        mesh=mesh,
        out_type=out_t,
        compiler_params=pltpu.CompilerParams(use_tc_tiling_on_sc=False),
        scratch_types=[
            pltpu.VMEM((_BPW,), jnp.int32),
            pltpu.VMEM((_BPW, _D), jnp.float32),
            pltpu.SemaphoreType.DMA,
        ],
    )
    def k(users_h, pos_h, neg_h, heads_h, rel_h, pt_h, nt_h,
          ue_h, ie_h, ke_h, re_h,
          u_o, pi_o, pkg_o, ni_o, nkg_o, h_o, pt_o, nt_o, r_o,
          idx_v, rows_v, sem):
        wid = lax.axis_index("s") * _NC + lax.axis_index("c")
        base = wid * _BPW
        jobs = (
            (users_h, ue_h, u_o),
            (pos_h, ie_h, pi_o),
            (pos_h, ke_h, pkg_o),
            (neg_h, ie_h, ni_o),
            (neg_h, ke_h, nkg_o),
            (heads_h, ke_h, h_o),
            (pt_h, ke_h, pt_o),
            (nt_h, ke_h, nt_o),
            (rel_h, re_h, r_o),
        )
        for idx_h, tab, out in jobs:
            pltpu.sync_copy(idx_h.at[pl.ds(base, _BPW)], idx_v)
            pltpu.async_copy(tab.at[idx_v], rows_v, sem).wait()
            pltpu.sync_copy(rows_v, out.at[pl.ds(base, _BPW)])

    return k(users, pos_items, neg_items, heads, relations, pos_tails,
             neg_tails, user_embed, item_embed, kg_entity_embed,
             kg_relation_embed)


def _l2n(x):
    n = jnp.sqrt(jnp.sum(x * x, axis=1, keepdims=True))
    return x / jnp.maximum(n, 1e-12)


def _tc_body(u_ref, pi_ref, pkg_ref, ni_ref, nkg_ref, h_ref, ptr_ref, ntr_ref,
             r_ref, rel_ref, w2_ref,
             pc_ref, nc_ref, hn_ref, rn_ref, ptn_ref, ntn_ref, pred_ref):
    i = pl.program_id(0)
    sl = pl.ds(i * _BB, _BB)

    pos_comb_full = pi_ref[...] + pkg_ref[...]
    pc_ref[...] = pi_ref[sl, :] + pkg_ref[sl, :]
    nc_ref[...] = ni_ref[...] + nkg_ref[...]

    pred_ref[...] = lax.dot_general(
        u_ref[...], pos_comb_full,
        dimension_numbers=(((1,), (1,)), ((), ())),
        preferred_element_type=jnp.float32)

    rd = _R * _D
    rel = rel_ref[...]
    lane = lax.broadcasted_iota(jnp.int32, (_BB, rd), 1)
    oh_exp = (jnp.broadcast_to(rel, (_BB, rd)) ==
              (lane // _D)).astype(jnp.float32)

    ic = lax.broadcasted_iota(jnp.int32, (rd, _D), 0)
    io = lax.broadcasted_iota(jnp.int32, (rd, _D), 1)
    sel = ((ic % _D) == io).astype(jnp.float32)

    w2 = w2_ref[...]
    for x_ref, out_ref in ((h_ref, hn_ref), (ptr_ref, ptn_ref),
                           (ntr_ref, ntn_ref)):
        y = jnp.dot(x_ref[...], w2,
                    preferred_element_type=jnp.float32)
        proj = jnp.dot(y * oh_exp, sel,
                       preferred_element_type=jnp.float32)
        out_ref[...] = _l2n(proj)

    rn_ref[...] = _l2n(r_ref[...])


def _tc_dense(u_e, pi, pkg, ni, nkg, h_raw, pt_raw, nt_raw, r_raw, rel2d, w2):
    blk = pl.BlockSpec((_BB, _D), lambda i: (i, 0))
    full = pl.BlockSpec((_B, _D), lambda i: (0, 0))
    return pl.pallas_call(
        _tc_body,
        grid=(_G,),
        in_specs=[
            blk,
            full,
            full,
            blk, blk,
            blk, blk, blk,
            blk,
            pl.BlockSpec((_BB, 1), lambda i: (i, 0)),
            pl.BlockSpec((_D, _R * _D), lambda i: (0, 0)),
        ],
        out_specs=[
            blk, blk, blk, blk, blk, blk,
            pl.BlockSpec((_BB, _B), lambda i: (i, 0)),
        ],
        out_shape=[
            jax.ShapeDtypeStruct((_B, _D), jnp.float32),
            jax.ShapeDtypeStruct((_B, _D), jnp.float32),
            jax.ShapeDtypeStruct((_B, _D), jnp.float32),
            jax.ShapeDtypeStruct((_B, _D), jnp.float32),
            jax.ShapeDtypeStruct((_B, _D), jnp.float32),
            jax.ShapeDtypeStruct((_B, _D), jnp.float32),
            jax.ShapeDtypeStruct((_B, _B), jnp.float32),
        ],
    )(u_e, pi, pkg, ni, nkg, h_raw, pt_raw, nt_raw, r_raw, rel2d, w2)


def kernel(users, pos_items, neg_items, heads, relations, pos_tails, neg_tails,
           user_embed, item_embed, kg_entity_embed, kg_relation_embed,
           trans_W):
    u_e, pi, pkg, ni, nkg, h_raw, pt_raw, nt_raw, r_raw = _sc_gather(
        users, pos_items, neg_items, heads, relations, pos_tails, neg_tails,
        user_embed, item_embed, kg_entity_embed, kg_relation_embed)
    w2 = jnp.transpose(trans_W, (1, 0, 2)).reshape(_D, _R * _D)
    rel2d = relations.reshape(_B, 1)
    pos_comb, neg_comb, h_n, r_n, pt_n, nt_n, preds = _tc_dense(
        u_e, pi, pkg, ni, nkg, h_raw, pt_raw, nt_raw, r_raw, rel2d, w2)
    return (u_e, pos_comb, neg_comb, h_n, r_n, pt_n, nt_n, preds)
